# baseline (speedup 1.0000x reference)
import functools

import jax
import jax.numpy as jnp
from jax import lax
from jax.experimental import pallas as pl
from jax.experimental.pallas import tpu as pltpu
from jax.experimental.pallas import tpu_sc as plsc

N_FIELDS = 26
VOCAB = 100000
BATCH = 16384

NW = 32
BCHUNK = BATCH // NW
HALF = N_FIELDS // 2

_mesh = plsc.VectorSubcoreMesh(core_axis_name="c", subcore_axis_name="s")


@functools.partial(
    pl.kernel,
    mesh=_mesh,
    out_type=jax.ShapeDtypeStruct((N_FIELDS, BATCH), jnp.float32),
    scratch_types=(
        [pltpu.VMEM((BCHUNK,), jnp.int32) for _ in range(N_FIELDS)]
        + [pltpu.VMEM((BCHUNK,), jnp.float32) for _ in range(N_FIELDS)]
        + [pltpu.SemaphoreType.DMA, pltpu.SemaphoreType.DMA, pltpu.SemaphoreType.DMA]
    ),
)
def _fm_gather(idx_hbm, ta_hbm, tb_hbm, out_hbm, *refs):
    idx_v = refs[:N_FIELDS]
    vals_v = refs[N_FIELDS:2 * N_FIELDS]
    isem, gsem, osem = refs[2 * N_FIELDS:]
    wid = lax.axis_index("s") * 2 + lax.axis_index("c")
    col0 = wid * BCHUNK

    loads = [
        pltpu.async_copy(idx_hbm.at[f, pl.ds(col0, BCHUNK)], idx_v[f], isem)
        for f in range(N_FIELDS)
    ]
    for c in loads:
        c.wait()
    gathers = [
        pltpu.async_copy(
            (ta_hbm if f < HALF else tb_hbm)
            .at[pl.ds((f % HALF) * VOCAB, VOCAB)]
            .at[idx_v[f]],
            vals_v[f],
            gsem,
        )
        for f in range(N_FIELDS)
    ]
    for g in gathers:
        g.wait()
    stores = [
        pltpu.async_copy(vals_v[f], out_hbm.at[f, pl.ds(col0, BCHUNK)], osem)
        for f in range(N_FIELDS)
    ]
    for c in stores:
        c.wait()


def kernel(data_batch, tables):
    idx_t = data_batch.astype(jnp.int32).T
    ta = lax.optimization_barrier(tables[:HALF, :, 0]).reshape(HALF * VOCAB)
    tb = lax.optimization_barrier(tables[HALF:, :, 0]).reshape(HALF * VOCAB)
    out_t = _fm_gather(idx_t, ta, tb)
    return out_t.T


# speedup vs baseline: 1.0055x; 1.0055x over previous
import functools

import jax
import jax.numpy as jnp
from jax import lax
from jax.experimental import pallas as pl
from jax.experimental.pallas import tpu as pltpu
from jax.experimental.pallas import tpu_sc as plsc

N_FIELDS = 26
VOCAB = 100000
BATCH = 16384

NW = 32
BCHUNK = BATCH // NW

_mesh = plsc.VectorSubcoreMesh(core_axis_name="c", subcore_axis_name="s")


@functools.partial(
    pl.kernel,
    mesh=_mesh,
    out_type=jax.ShapeDtypeStruct((N_FIELDS, BATCH), jnp.float32),
    scratch_types=(
        [pltpu.VMEM((BCHUNK,), jnp.int32) for _ in range(N_FIELDS)]
        + [pltpu.VMEM((BCHUNK,), jnp.float32) for _ in range(N_FIELDS)]
        + [pltpu.SemaphoreType.DMA, pltpu.SemaphoreType.DMA, pltpu.SemaphoreType.DMA]
    ),
)
def _fm_gather(idx_hbm, table_hbm, out_hbm, *refs):
    idx_v = refs[:N_FIELDS]
    vals_v = refs[N_FIELDS:2 * N_FIELDS]
    isem, gsem, osem = refs[2 * N_FIELDS:]
    wid = lax.axis_index("s") * 2 + lax.axis_index("c")
    col0 = wid * BCHUNK

    loads = [
        pltpu.async_copy(idx_hbm.at[f, pl.ds(col0, BCHUNK)], idx_v[f], isem)
        for f in range(N_FIELDS)
    ]
    for c in loads:
        c.wait()
    gathers = [
        pltpu.async_copy(
            table_hbm.at[pl.ds(f * VOCAB, VOCAB)].at[idx_v[f]], vals_v[f], gsem
        )
        for f in range(N_FIELDS)
    ]
    for g in gathers:
        g.wait()
    stores = [
        pltpu.async_copy(vals_v[f], out_hbm.at[f, pl.ds(col0, BCHUNK)], osem)
        for f in range(N_FIELDS)
    ]
    for c in stores:
        c.wait()


def kernel(data_batch, tables):
    idx_t = data_batch.astype(jnp.int32).T
    table_flat = lax.optimization_barrier(tables[:, :, 0]).reshape(N_FIELDS * VOCAB)
    out_t = _fm_gather(idx_t, table_flat)
    return out_t.T


# speedup vs baseline: 1.0073x; 1.0019x over previous
import functools

import jax
import jax.numpy as jnp
from jax import lax
from jax.experimental import pallas as pl
from jax.experimental.pallas import tpu as pltpu
from jax.experimental.pallas import tpu_sc as plsc

N_FIELDS = 26
VOCAB = 100000
BATCH = 16384

NW = 32
BCHUNK = BATCH // NW
HALF = N_FIELDS // 2

_mesh = plsc.VectorSubcoreMesh(core_axis_name="c", subcore_axis_name="s")


@functools.partial(
    pl.kernel,
    mesh=_mesh,
    out_type=jax.ShapeDtypeStruct((N_FIELDS, BATCH), jnp.float32),
    scratch_types=(
        [pltpu.VMEM((BCHUNK,), jnp.int32) for _ in range(N_FIELDS)]
        + [pltpu.VMEM((BCHUNK,), jnp.float32) for _ in range(N_FIELDS)]
        + [
            pltpu.SemaphoreType.DMA,
            pltpu.SemaphoreType.DMA,
            pltpu.SemaphoreType.DMA,
            pltpu.SemaphoreType.DMA,
        ]
    ),
)
def _fm_gather(idx_hbm, table_hbm, out_hbm, *refs):
    idx_v = refs[:N_FIELDS]
    vals_v = refs[N_FIELDS:2 * N_FIELDS]
    isem, gsem_a, gsem_b, osem = refs[2 * N_FIELDS:]
    wid = lax.axis_index("s") * 2 + lax.axis_index("c")
    col0 = wid * BCHUNK

    loads = [
        pltpu.async_copy(idx_hbm.at[f, pl.ds(col0, BCHUNK)], idx_v[f], isem)
        for f in range(N_FIELDS)
    ]
    for c in loads:
        c.wait()

    def gather(f, sem):
        return pltpu.async_copy(
            table_hbm.at[pl.ds(f * VOCAB, VOCAB)].at[idx_v[f]], vals_v[f], sem
        )

    def store(f):
        return pltpu.async_copy(vals_v[f], out_hbm.at[f, pl.ds(col0, BCHUNK)], osem)

    ga = [gather(f, gsem_a) for f in range(HALF)]
    gb = [gather(f, gsem_b) for f in range(HALF, N_FIELDS)]
    for g in ga:
        g.wait()
    sa = [store(f) for f in range(HALF)]
    for g in gb:
        g.wait()
    sb = [store(f) for f in range(HALF, N_FIELDS)]
    for c in sa + sb:
        c.wait()


def kernel(data_batch, tables):
    idx_t = data_batch.astype(jnp.int32).T
    table_flat = lax.optimization_barrier(tables[:, :, 0]).reshape(N_FIELDS * VOCAB)
    out_t = _fm_gather(idx_t, table_flat)
    return out_t.T
